# baseline (speedup 1.0000x reference)
import functools

import jax
import jax.numpy as jnp
from jax import lax
from jax.experimental import pallas as pl
from jax.experimental.pallas import tpu as pltpu
from jax.experimental.pallas import tpu_sc as plsc

VOCAB = 1000000
EMBED = 16
BATCH = 4096

_NC = 2
_NS = 16
_NW = _NC * _NS
_BPW = BATCH // _NW
_L = 16
_GRP = _BPW // _L


def _sc_gather_kernel():
    mesh = plsc.VectorSubcoreMesh(core_axis_name="c", subcore_axis_name="s")

    @functools.partial(
        pl.kernel,
        mesh=mesh,
        compiler_params=pltpu.CompilerParams(needs_layout_passes=False),
        out_type=(
            jax.ShapeDtypeStruct((BATCH, 128), jnp.float32),
            jax.ShapeDtypeStruct((BATCH, 128), jnp.float32),
        ),
        scratch_types=[
            pltpu.VMEM((_BPW,), jnp.int32),
            pltpu.VMEM((_BPW,), jnp.int32),
            pltpu.VMEM((_L, 2, 8, 128), jnp.float32),
            pltpu.VMEM((_L, 2, 8, 128), jnp.float32),
            pltpu.VMEM((_BPW, 128), jnp.float32),
            pltpu.VMEM((_BPW, 128), jnp.float32),
            pltpu.SemaphoreType.DMA,
            pltpu.SemaphoreType.DMA,
        ],
    )
    def gather(cw_hbm, xw_hbm, vt_hbm, ut_hbm, outv_hbm, outu_hbm,
               idx_c, idx_x, buf0, buf1, out_c, out_x, sem0, sem1):
        wid = lax.axis_index("s") * _NC + lax.axis_index("c")
        base = wid * _BPW
        pltpu.sync_copy(cw_hbm.at[pl.ds(base, _BPW)], idx_c)
        pltpu.sync_copy(xw_hbm.at[pl.ds(base, _BPW)], idx_x)

        lanes = lax.iota(jnp.int32, _L)
        bufs = (buf0, buf1)
        sems = (sem0, sem1)

        steps = [(vt_hbm, idx_c, out_c, g) for g in range(_GRP)]
        steps += [(ut_hbm, idx_x, out_x, g) for g in range(_GRP)]

        def issue(step, slot):
            table_hbm, idx, _, g = step
            gran = idx[pl.ds(g * _L, _L)] >> 7
            cps = []
            for k in range(_L):
                c = jnp.max(jnp.where(lanes == k, gran, 0))
                start = pl.multiple_of(c * 128, 128)
                cps.append(pltpu.async_copy(
                    table_hbm.at[:, :, pl.ds(start, 128)],
                    bufs[slot].at[k], sems[slot]))
            return cps

        def extract(step, slot, cps):
            _, idx, out, g = step
            for cp in cps:
                cp.wait()
            lvec = idx[pl.ds(g * _L, _L)] & 127
            rowv = g * _L + lanes
            for e in range(EMBED):
                val = plsc.load_gather(
                    bufs[slot],
                    [lanes, jnp.full((_L,), e // 8, jnp.int32),
                     jnp.full((_L,), e % 8, jnp.int32), lvec])
                plsc.store_scatter(
                    out, [rowv, jnp.full((_L,), e, jnp.int32)], val)

        pending = issue(steps[0], 0)
        for s in range(len(steps)):
            nxt = None
            if s + 1 < len(steps):
                nxt = issue(steps[s + 1], (s + 1) % 2)
            extract(steps[s], s % 2, pending)
            pending = nxt

        pltpu.sync_copy(out_c, outv_hbm.at[pl.ds(base, _BPW)])
        pltpu.sync_copy(out_x, outu_hbm.at[pl.ds(base, _BPW)])

    return gather


_ROW_TILE = 512


def _score_softmax_body(c_ref, x_ref, o_ref):
    scores = lax.dot_general(
        c_ref[:, :EMBED], x_ref[:, :EMBED],
        dimension_numbers=(((1,), (1,)), ((), ())),
        preferred_element_type=jnp.float32,
    )
    m = jnp.max(scores, axis=1, keepdims=True)
    e = jnp.exp(scores - m)
    s = jnp.sum(e, axis=1, keepdims=True)
    o_ref[...] = scores - (m + jnp.log(s))


def kernel(center_words, context_words, embedding_v, embedding_u):
    vt = embedding_v.T.reshape(2, 8, VOCAB)
    ut = embedding_u.T.reshape(2, 8, VOCAB)
    center_embed, context_embed = _sc_gather_kernel()(
        center_words.astype(jnp.int32), context_words.astype(jnp.int32),
        vt, ut)

    log_probs = pl.pallas_call(
        _score_softmax_body,
        grid=(BATCH // _ROW_TILE,),
        in_specs=[
            pl.BlockSpec((_ROW_TILE, 128), lambda i: (i, 0)),
            pl.BlockSpec((BATCH, 128), lambda i: (0, 0)),
        ],
        out_specs=pl.BlockSpec((_ROW_TILE, BATCH), lambda i: (i, 0)),
        out_shape=jax.ShapeDtypeStruct((BATCH, BATCH), jnp.float32),
    )(center_embed, context_embed)
    return log_probs
